# baseline (speedup 1.0000x reference)
import functools

import jax
import jax.numpy as jnp
from jax import lax
from jax.experimental import pallas as pl
from jax.experimental.pallas import tpu as pltpu
from jax.experimental.pallas import tpu_sc as plsc

NW = 32
LANES = 16
CHUNK = 128
NCH = 2


def _make_row_gather(n_rows: int, d: int, dtype):
    mesh = plsc.VectorSubcoreMesh(core_axis_name="c", subcore_axis_name="s")
    bpw = n_rows // NW
    ch = min(CHUNK, bpw)
    nch = bpw // ch

    @functools.partial(
        pl.kernel,
        mesh=mesh,
        out_type=jax.ShapeDtypeStruct((n_rows, d), dtype),
        scratch_types=[
            pltpu.VMEM((bpw,), jnp.int32),
            pltpu.VMEM((ch, d), dtype),
            pltpu.VMEM((ch, d), dtype),
            pltpu.SemaphoreType.DMA,
            pltpu.SemaphoreType.DMA,
            pltpu.SemaphoreType.DMA,
            pltpu.SemaphoreType.DMA,
        ],
    )
    def gather(src_hbm, idx_hbm, out_hbm, idx_v, r0, r1, gs0, gs1, ws0, ws1):
        wid = lax.axis_index("s") * 2 + lax.axis_index("c")
        base = wid * bpw
        pltpu.sync_copy(idx_hbm.at[pl.ds(base, bpw)], idx_v)
        bufs = (r0, r1)
        gsem = (gs0, gs1)
        wsem = (ws0, ws1)
        gh = [None] * nch
        wh = [None] * nch
        for c in range(nch):
            if c >= 2:
                wh[c - 2].wait()
            gh[c] = pltpu.async_copy(
                src_hbm.at[idx_v.at[pl.ds(c * ch, ch)]], bufs[c % 2],
                gsem[c % 2])
            if c >= 1:
                gh[c - 1].wait()
                wh[c - 1] = pltpu.async_copy(
                    bufs[(c - 1) % 2],
                    out_hbm.at[pl.ds(base + (c - 1) * ch, ch)],
                    wsem[(c - 1) % 2])
        gh[nch - 1].wait()
        wh[nch - 1] = pltpu.async_copy(
            bufs[(nch - 1) % 2],
            out_hbm.at[pl.ds(base + (nch - 1) * ch, ch)], wsem[(nch - 1) % 2])
        if nch >= 2:
            wh[nch - 2].wait()
        wh[nch - 1].wait()

    return gather


def _make_row_scatter(n_rows: int, d: int, dtype, n_chunks: int):
    mesh = plsc.VectorSubcoreMesh(core_axis_name="c", subcore_axis_name="s")
    bpw = n_rows // NW
    nch = bpw // CHUNK
    tiles_per_chunk = NW // n_chunks

    ch = 64
    nck = bpw // ch

    @functools.partial(
        pl.kernel,
        mesh=mesh,
        out_type=jax.ShapeDtypeStruct((n_rows, d), dtype),
        scratch_types=(
            [pltpu.VMEM((ch,), jnp.int32) for _ in range(nck)]
            + [pltpu.VMEM((ch, d), dtype) for _ in range(2)]
            + [pltpu.SemaphoreType.DMA for _ in range(4)]
        ),
    )
    def scatter(*args):
        srcs = args[:n_chunks]
        idx_hbm = args[n_chunks]
        out_hbm = args[n_chunks + 1]
        rest = args[n_chunks + 2:]
        idx_refs = rest[:nck]
        bufs = rest[nck:nck + 2]
        rsem = rest[nck + 2:nck + 4]
        ssem = rest[nck + 4:nck + 6]
        wid = lax.axis_index("s") * 2 + lax.axis_index("c")
        base = wid * bpw
        for c in range(nck):
            pltpu.sync_copy(idx_hbm.at[pl.ds(base + c * ch, ch)], idx_refs[c])
        for k in range(n_chunks):
            lo = k * tiles_per_chunk
            @pl.when((wid >= lo) & (wid < lo + tiles_per_chunk))
            def _():
                local = (wid - lo) * bpw
                rh = [None] * nck
                sh = [None] * nck
                for c in range(nck):
                    if c >= 2:
                        sh[c - 2].wait()
                    rh[c] = pltpu.async_copy(
                        srcs[k].at[pl.ds(local + c * ch, ch)], bufs[c % 2],
                        rsem[c % 2])
                    if c >= 1:
                        rh[c - 1].wait()
                        sh[c - 1] = pltpu.async_copy(
                            bufs[(c - 1) % 2], out_hbm.at[idx_refs[c - 1]],
                            ssem[(c - 1) % 2])
                rh[nck - 1].wait()
                sh[nck - 1] = pltpu.async_copy(
                    bufs[(nck - 1) % 2], out_hbm.at[idx_refs[nck - 1]],
                    ssem[(nck - 1) % 2])
                if nck >= 2:
                    sh[nck - 2].wait()
                sh[nck - 1].wait()

    return scatter


def _gmm_body(ends_ref, xs_ref, w_ref, b_ref, out_ref, *, block_rows, n_exp,
              row_base):
    i = pl.program_id(0)
    row0 = row_base + i * block_rows
    ridx = row0 + lax.broadcasted_iota(jnp.int32, (block_rows, 1), 0)
    e_row = jnp.zeros((block_rows, 1), jnp.int32)
    e_lo = jnp.int32(0)
    e_hi = jnp.int32(0)
    for e in range(n_exp - 1):
        end_e = ends_ref[e]
        e_row = e_row + (ridx >= end_e).astype(jnp.int32)
        e_lo = e_lo + (row0 >= end_e).astype(jnp.int32)
        e_hi = e_hi + (row0 + block_rows - 1 >= end_e).astype(jnp.int32)

    x = xs_ref[:]

    def body(e, _):
        y = jnp.dot(x, w_ref[e], preferred_element_type=jnp.float32)
        y = y + b_ref[e]
        out_ref[:] = jnp.where(e_row == e, y, out_ref[:])
        return 0

    out_ref[:] = jnp.zeros_like(out_ref)
    lax.fori_loop(e_lo, e_hi + 1, body, 0)


def _grouped_matmul(ends, xs, w, b3, block_rows: int, row_base: int):
    rows, d_in = xs.shape
    n_exp, _, d_out = w.shape
    grid = (rows // block_rows,)
    grid_spec = pltpu.PrefetchScalarGridSpec(
        num_scalar_prefetch=1,
        grid=grid,
        in_specs=[
            pl.BlockSpec((block_rows, d_in), lambda i, ends: (i, 0)),
            pl.BlockSpec((n_exp, d_in, d_out), lambda i, ends: (0, 0, 0)),
            pl.BlockSpec((n_exp, 1, d_out), lambda i, ends: (0, 0, 0)),
        ],
        out_specs=pl.BlockSpec((block_rows, d_out), lambda i, ends: (i, 0)),
    )
    return pl.pallas_call(
        functools.partial(_gmm_body, block_rows=block_rows, n_exp=n_exp,
                          row_base=row_base),
        grid_spec=grid_spec,
        out_shape=jax.ShapeDtypeStruct((rows, d_out), jnp.float32),
        compiler_params=pltpu.CompilerParams(
            dimension_semantics=("arbitrary",),
        ),
    )(ends, xs, w, b3)


def kernel(X, D, W, b):
    n, d_in = X.shape
    n_exp, _, d_out = W.shape
    rows_per_chunk = n // NCH

    key = D.astype(jnp.int32) * n + jnp.arange(n, dtype=jnp.int32)
    skey = jnp.sort(key)
    perm = skey % n
    ends = jnp.searchsorted(
        skey, (jnp.arange(n_exp, dtype=jnp.int32) + 1) * n).astype(jnp.int32)

    gather = _make_row_gather(rows_per_chunk, d_in, X.dtype)
    b3 = b.reshape(n_exp, 1, d_out)
    ys = []
    for k in range(NCH):
        perm_k = lax.dynamic_slice_in_dim(perm, k * rows_per_chunk,
                                          rows_per_chunk)
        xs_k = gather(X, perm_k)
        ys.append(_grouped_matmul(ends, xs_k, W, b3, block_rows=512,
                                  row_base=k * rows_per_chunk))

    scatter = _make_row_scatter(n, d_out, jnp.float32, NCH)
    theta = scatter(*ys, perm)
    return theta


# speedup vs baseline: 1.0147x; 1.0147x over previous
import functools

import jax
import jax.numpy as jnp
from jax import lax
from jax.experimental import pallas as pl
from jax.experimental.pallas import tpu as pltpu
from jax.experimental.pallas import tpu_sc as plsc

NW = 32
LANES = 16
CHUNK = 128
NCH = 2


def _make_row_gather(n_rows: int, d: int, dtype):
    mesh = plsc.VectorSubcoreMesh(core_axis_name="c", subcore_axis_name="s")
    bpw = n_rows // NW
    ch = min(CHUNK, bpw)
    nch = bpw // ch

    @functools.partial(
        pl.kernel,
        mesh=mesh,
        out_type=jax.ShapeDtypeStruct((n_rows, d), dtype),
        scratch_types=[
            pltpu.VMEM((bpw,), jnp.int32),
            pltpu.VMEM((ch, d), dtype),
            pltpu.VMEM((ch, d), dtype),
            pltpu.SemaphoreType.DMA,
            pltpu.SemaphoreType.DMA,
            pltpu.SemaphoreType.DMA,
            pltpu.SemaphoreType.DMA,
        ],
    )
    def gather(src_hbm, idx_hbm, out_hbm, idx_v, r0, r1, gs0, gs1, ws0, ws1):
        wid = lax.axis_index("s") * 2 + lax.axis_index("c")
        base = wid * bpw
        pltpu.sync_copy(idx_hbm.at[pl.ds(base, bpw)], idx_v)
        bufs = (r0, r1)
        gsem = (gs0, gs1)
        wsem = (ws0, ws1)
        gh = [None] * nch
        wh = [None] * nch
        for c in range(nch):
            if c >= 2:
                wh[c - 2].wait()
            gh[c] = pltpu.async_copy(
                src_hbm.at[idx_v.at[pl.ds(c * ch, ch)]], bufs[c % 2],
                gsem[c % 2])
            if c >= 1:
                gh[c - 1].wait()
                wh[c - 1] = pltpu.async_copy(
                    bufs[(c - 1) % 2],
                    out_hbm.at[pl.ds(base + (c - 1) * ch, ch)],
                    wsem[(c - 1) % 2])
        gh[nch - 1].wait()
        wh[nch - 1] = pltpu.async_copy(
            bufs[(nch - 1) % 2],
            out_hbm.at[pl.ds(base + (nch - 1) * ch, ch)], wsem[(nch - 1) % 2])
        if nch >= 2:
            wh[nch - 2].wait()
        wh[nch - 1].wait()

    return gather


def _make_row_scatter(n_rows: int, d: int, dtype, n_chunks: int):
    mesh = plsc.VectorSubcoreMesh(core_axis_name="c", subcore_axis_name="s")
    bpw = n_rows // NW
    nch = bpw // CHUNK
    tiles_per_chunk = NW // n_chunks

    @functools.partial(
        pl.kernel,
        mesh=mesh,
        out_type=jax.ShapeDtypeStruct((n_rows, d), dtype),
        scratch_types=[
            pltpu.VMEM((CHUNK,), jnp.int32),
            pltpu.VMEM((CHUNK, d), dtype),
            pltpu.SemaphoreType.DMA,
        ],
    )
    def scatter(*args):
        srcs = args[:n_chunks]
        idx_hbm = args[n_chunks]
        out_hbm = args[n_chunks + 1]
        idx_v, rows_v, sem = args[n_chunks + 2:]
        wid = lax.axis_index("s") * 2 + lax.axis_index("c")
        for k in range(n_chunks):
            lo = k * tiles_per_chunk
            @pl.when((wid >= lo) & (wid < lo + tiles_per_chunk))
            def _():
                for c in range(nch):
                    base = wid * bpw + c * CHUNK
                    local = (wid - lo) * bpw + c * CHUNK
                    pltpu.sync_copy(idx_hbm.at[pl.ds(base, CHUNK)], idx_v)
                    pltpu.sync_copy(srcs[k].at[pl.ds(local, CHUNK)], rows_v)
                    pltpu.async_copy(rows_v, out_hbm.at[idx_v], sem).wait()

    return scatter


def _gmm_body(ends_ref, xs_ref, w_ref, b_ref, out_ref, *, block_rows, n_exp,
              row_base):
    i = pl.program_id(0)
    row0 = row_base + i * block_rows
    ridx = row0 + lax.broadcasted_iota(jnp.int32, (block_rows, 1), 0)
    e_row = jnp.zeros((block_rows, 1), jnp.int32)
    e_lo = jnp.int32(0)
    e_hi = jnp.int32(0)
    for e in range(n_exp - 1):
        end_e = ends_ref[e]
        e_row = e_row + (ridx >= end_e).astype(jnp.int32)
        e_lo = e_lo + (row0 >= end_e).astype(jnp.int32)
        e_hi = e_hi + (row0 + block_rows - 1 >= end_e).astype(jnp.int32)

    x = xs_ref[:]

    def body(e, _):
        y = jnp.dot(x, w_ref[e], preferred_element_type=jnp.float32)
        y = y + b_ref[e]
        out_ref[:] = jnp.where(e_row == e, y, out_ref[:])
        return 0

    out_ref[:] = jnp.zeros_like(out_ref)
    lax.fori_loop(e_lo, e_hi + 1, body, 0)


def _grouped_matmul(ends, xs, w, b3, block_rows: int, row_base: int):
    rows, d_in = xs.shape
    n_exp, _, d_out = w.shape
    grid = (rows // block_rows,)
    grid_spec = pltpu.PrefetchScalarGridSpec(
        num_scalar_prefetch=1,
        grid=grid,
        in_specs=[
            pl.BlockSpec((block_rows, d_in), lambda i, ends: (i, 0)),
            pl.BlockSpec((n_exp, d_in, d_out), lambda i, ends: (0, 0, 0)),
            pl.BlockSpec((n_exp, 1, d_out), lambda i, ends: (0, 0, 0)),
        ],
        out_specs=pl.BlockSpec((block_rows, d_out), lambda i, ends: (i, 0)),
    )
    return pl.pallas_call(
        functools.partial(_gmm_body, block_rows=block_rows, n_exp=n_exp,
                          row_base=row_base),
        grid_spec=grid_spec,
        out_shape=jax.ShapeDtypeStruct((rows, d_out), jnp.float32),
        compiler_params=pltpu.CompilerParams(
            dimension_semantics=("arbitrary",),
        ),
    )(ends, xs, w, b3)


def kernel(X, D, W, b):
    n, d_in = X.shape
    n_exp, _, d_out = W.shape
    rows_per_chunk = n // NCH

    key = D.astype(jnp.int32) * n + jnp.arange(n, dtype=jnp.int32)
    skey = jnp.sort(key)
    perm = skey % n
    ends = jnp.searchsorted(
        skey, (jnp.arange(n_exp, dtype=jnp.int32) + 1) * n).astype(jnp.int32)

    gather = _make_row_gather(rows_per_chunk, d_in, X.dtype)
    b3 = b.reshape(n_exp, 1, d_out)
    ys = []
    for k in range(NCH):
        perm_k = lax.dynamic_slice_in_dim(perm, k * rows_per_chunk,
                                          rows_per_chunk)
        xs_k = gather(X, perm_k)
        ys.append(_grouped_matmul(ends, xs_k, W, b3, block_rows=512,
                                  row_base=k * rows_per_chunk))

    scatter = _make_row_scatter(n, d_out, jnp.float32, NCH)
    theta = scatter(*ys, perm)
    return theta
